# baseline (speedup 1.0000x reference)
import functools
import jax
import jax.numpy as jnp
from jax import lax
from jax.experimental import pallas as pl
from jax.experimental.pallas import tpu as pltpu
from jax.experimental.pallas import tpu_sc as plsc

NC = 2
NS = 16
NW = NC * NS
LANES = 16

D = 32
PADW = 128
GBUF = 8
SGRP = 4
SBUF = 2


def _body(w_hbm, idx_hbm, bias_hbm, out_hbm, idx_v, bias_v, rows_v, stage_v,
          gsem, ssem, *, bpw, hist):
    cid = lax.axis_index("c")
    sid = lax.axis_index("s")
    wid = sid * NC + cid
    grows = SGRP * hist

    pltpu.sync_copy(idx_hbm.at[pl.ds(wid * bpw * PADW, bpw * PADW)], idx_v)
    pltpu.sync_copy(bias_hbm, bias_v)
    b0 = bias_v[pl.ds(0, LANES)]
    b1 = bias_v[pl.ds(LANES, LANES)]

    def fire_gather(c, slot):
        pltpu.async_copy(w_hbm.at[idx_v.at[pl.ds(c * PADW, hist)]],
                         rows_v.at[slot], gsem.at[slot])

    def wait_gather(c, slot):
        pltpu.make_async_copy(w_hbm.at[idx_v.at[pl.ds(c * PADW, hist)]],
                              rows_v.at[slot], gsem.at[slot]).wait()

    def out_rows(grp):
        return pl.ds(wid * bpw * hist + grp * grows, grows)

    def fire_store(grp, slot):
        pltpu.async_copy(stage_v.at[slot], out_hbm.at[out_rows(grp)],
                         ssem.at[slot])

    def wait_store(grp, slot):
        pltpu.make_async_copy(stage_v.at[slot], out_hbm.at[out_rows(grp)],
                              ssem.at[slot]).wait()

    for b in range(GBUF):
        fire_gather(b, b)

    def iter_body(g, carry):
        gslot = g % GBUF
        grp = g // SGRP
        r = g % SGRP
        sslot = grp % SBUF

        @pl.when(jnp.logical_and(r == 0, grp >= SBUF))
        def _():
            wait_store(grp - SBUF, sslot)

        wait_gather(g, gslot)

        @plsc.parallel_loop(0, hist, unroll=10)
        def _(i):
            stage_v[sslot, r * hist + i, pl.ds(0, LANES)] = (
                rows_v[gslot, i, pl.ds(0, LANES)] + b0)
            stage_v[sslot, r * hist + i, pl.ds(LANES, LANES)] = (
                rows_v[gslot, i, pl.ds(LANES, LANES)] + b1)

        @pl.when(g + GBUF < bpw)
        def _():
            fire_gather(g + GBUF, gslot)

        @pl.when(r == SGRP - 1)
        def _():
            fire_store(grp, sslot)

        return carry

    lax.fori_loop(0, bpw, iter_body, 0)

    ngrp = bpw // SGRP
    for b in range(SBUF):
        grp = ngrp - SBUF + b
        wait_store(grp, grp % SBUF)


def kernel(input, weight, bias):
    idx = input.astype(jnp.int32)
    batch, hist = idx.shape
    n = batch * hist
    assert batch % (NW * SGRP) == 0
    bpw = batch // NW
    assert bpw >= GBUF and bpw % SGRP == 0

    idx_p = jnp.pad(idx, ((0, 0), (0, PADW - hist))).reshape(-1)
    w_p = jnp.pad(weight, ((0, 0), (0, PADW - D)))

    mesh = plsc.VectorSubcoreMesh(core_axis_name="c", subcore_axis_name="s")
    run = pl.kernel(
        functools.partial(_body, bpw=bpw, hist=hist),
        out_type=jax.ShapeDtypeStruct((n, PADW), jnp.float32),
        mesh=mesh,
        scratch_types=[
            pltpu.VMEM((bpw * PADW,), jnp.int32),
            pltpu.VMEM((D,), jnp.float32),
            pltpu.VMEM((GBUF, hist, PADW), jnp.float32),
            pltpu.VMEM((SBUF, SGRP * hist, PADW), jnp.float32),
            pltpu.SemaphoreType.DMA((GBUF,)),
            pltpu.SemaphoreType.DMA((SBUF,)),
        ],
        compiler_params=pltpu.CompilerParams(use_tc_tiling_on_sc=False),
    )
    res = run(w_p, idx_p, bias)
    return res[:, :D]


# speedup vs baseline: 1.0173x; 1.0173x over previous
import functools
import jax
import jax.numpy as jnp
from jax import lax
from jax.experimental import pallas as pl
from jax.experimental.pallas import tpu as pltpu
from jax.experimental.pallas import tpu_sc as plsc

NC = 2
NS = 16
NW = NC * NS
LANES = 16

D = 32
PADW = 128
CHUNK = 256
GBUF = 3
SBUF = 2


def _body(w_hbm, idx_hbm, bias_hbm, out_hbm, idx_v, cidx_v, bias_v, rows_v,
          stage_v, gsem, ssem, *, bpw, hist):
    cid = lax.axis_index("c")
    sid = lax.axis_index("s")
    wid = sid * NC + cid
    npw = bpw * hist
    cpw = npw // CHUNK

    pltpu.sync_copy(idx_hbm.at[pl.ds(wid * bpw * PADW, bpw * PADW)], idx_v)
    pltpu.sync_copy(bias_hbm, bias_v)
    b0 = bias_v[pl.ds(0, LANES)]
    b1 = bias_v[pl.ds(LANES, LANES)]

    iota = lax.iota(jnp.int32, LANES)

    def compact_body(k, carry):
        j = k * LANES + iota
        row = lax.shift_right_logical(j * 5243, 18)
        addr = row * (PADW - hist) + j
        cidx_v[pl.ds(k * LANES, LANES)] = plsc.load_gather(idx_v, [addr])
        return carry

    lax.fori_loop(0, npw // LANES, compact_body, 0)

    def fire_gather(c, slot):
        pltpu.async_copy(w_hbm.at[cidx_v.at[pl.ds(c * CHUNK, CHUNK)]],
                         rows_v.at[slot], gsem.at[slot])

    def wait_gather(c, slot):
        pltpu.make_async_copy(w_hbm.at[cidx_v.at[pl.ds(c * CHUNK, CHUNK)]],
                              rows_v.at[slot], gsem.at[slot]).wait()

    def out_rows(c):
        return pl.ds(wid * npw + c * CHUNK, CHUNK)

    def fire_store(c, slot):
        pltpu.async_copy(stage_v.at[slot], out_hbm.at[out_rows(c)],
                         ssem.at[slot])

    def wait_store(c, slot):
        pltpu.make_async_copy(stage_v.at[slot], out_hbm.at[out_rows(c)],
                              ssem.at[slot]).wait()

    for b in range(GBUF):
        fire_gather(b, b)

    def iter_body(g, carry):
        gslot = g % GBUF
        sslot = g % SBUF

        @pl.when(g >= SBUF)
        def _():
            wait_store(g - SBUF, sslot)

        wait_gather(g, gslot)

        @plsc.parallel_loop(0, CHUNK, unroll=8)
        def _(i):
            stage_v[sslot, i, pl.ds(0, LANES)] = (
                rows_v[gslot, i, pl.ds(0, LANES)] + b0)
            stage_v[sslot, i, pl.ds(LANES, LANES)] = (
                rows_v[gslot, i, pl.ds(LANES, LANES)] + b1)

        @pl.when(g + GBUF < cpw)
        def _():
            fire_gather(g + GBUF, gslot)

        fire_store(g, sslot)
        return carry

    lax.fori_loop(0, cpw, iter_body, 0)

    for b in range(SBUF):
        c = cpw - SBUF + b
        wait_store(c, c % SBUF)


def kernel(input, weight, bias):
    idx = input.astype(jnp.int32)
    batch, hist = idx.shape
    n = batch * hist
    assert batch % NW == 0
    bpw = batch // NW
    assert (bpw * hist) % CHUNK == 0 and (bpw * hist) // CHUNK >= GBUF

    idx_p = jnp.pad(idx, ((0, 0), (0, PADW - hist))).reshape(-1)

    mesh = plsc.VectorSubcoreMesh(core_axis_name="c", subcore_axis_name="s")
    run = pl.kernel(
        functools.partial(_body, bpw=bpw, hist=hist),
        out_type=jax.ShapeDtypeStruct((n, PADW), jnp.float32),
        mesh=mesh,
        scratch_types=[
            pltpu.VMEM((bpw * PADW,), jnp.int32),
            pltpu.VMEM((bpw * hist,), jnp.int32),
            pltpu.VMEM((D,), jnp.float32),
            pltpu.VMEM((GBUF, CHUNK, D), jnp.float32),
            pltpu.VMEM((SBUF, CHUNK, PADW), jnp.float32),
            pltpu.SemaphoreType.DMA((GBUF,)),
            pltpu.SemaphoreType.DMA((SBUF,)),
        ],
        compiler_params=pltpu.CompilerParams(
            use_tc_tiling_on_sc=False, needs_layout_passes=False),
    )
    res = run(weight, idx_p, bias)
    return res[:, :D]


# speedup vs baseline: 1.0185x; 1.0012x over previous
import functools
import jax
import jax.numpy as jnp
from jax import lax
from jax.experimental import pallas as pl
from jax.experimental.pallas import tpu as pltpu
from jax.experimental.pallas import tpu_sc as plsc

NC = 2
NS = 16
NW = NC * NS
LANES = 16

D = 32
PADW = 128
GBUF = 8
SGRP = 4
SBUF = 2


def _body(w_hbm, idx_hbm, bias_hbm, out_hbm, idx_v, bias_v, rows_v, stage_v,
          gsem, ssem, *, bpw, hist):
    cid = lax.axis_index("c")
    sid = lax.axis_index("s")
    wid = sid * NC + cid
    grows = SGRP * hist

    pltpu.sync_copy(idx_hbm.at[pl.ds(wid * bpw * PADW, bpw * PADW)], idx_v)
    pltpu.sync_copy(bias_hbm, bias_v)
    b0 = bias_v[pl.ds(0, LANES)]
    b1 = bias_v[pl.ds(LANES, LANES)]

    def fire_gather(c, slot):
        pltpu.async_copy(w_hbm.at[idx_v.at[pl.ds(c * PADW, hist)]],
                         rows_v.at[slot], gsem.at[slot])

    def wait_gather(c, slot):
        pltpu.make_async_copy(w_hbm.at[idx_v.at[pl.ds(c * PADW, hist)]],
                              rows_v.at[slot], gsem.at[slot]).wait()

    def out_rows(grp):
        return pl.ds(wid * bpw * hist + grp * grows, grows)

    def fire_store(grp, slot):
        pltpu.async_copy(stage_v.at[slot], out_hbm.at[out_rows(grp)],
                         ssem.at[slot])

    def wait_store(grp, slot):
        pltpu.make_async_copy(stage_v.at[slot], out_hbm.at[out_rows(grp)],
                              ssem.at[slot]).wait()

    for b in range(GBUF):
        fire_gather(b, b)

    def iter_body(g, carry):
        gslot = g % GBUF
        grp = g // SGRP
        r = g % SGRP
        sslot = grp % SBUF

        @pl.when(jnp.logical_and(r == 0, grp >= SBUF))
        def _():
            wait_store(grp - SBUF, sslot)

        wait_gather(g, gslot)

        @plsc.parallel_loop(0, hist, unroll=10)
        def _(i):
            stage_v[sslot, r * hist + i, pl.ds(0, LANES)] = (
                rows_v[gslot, i, pl.ds(0, LANES)] + b0)
            stage_v[sslot, r * hist + i, pl.ds(LANES, LANES)] = (
                rows_v[gslot, i, pl.ds(LANES, LANES)] + b1)

        @pl.when(g + GBUF < bpw)
        def _():
            fire_gather(g + GBUF, gslot)

        @pl.when(r == SGRP - 1)
        def _():
            fire_store(grp, sslot)

        return carry

    lax.fori_loop(0, bpw, iter_body, 0)

    ngrp = bpw // SGRP
    for b in range(SBUF):
        grp = ngrp - SBUF + b
        wait_store(grp, grp % SBUF)


def kernel(input, weight, bias):
    idx = input.astype(jnp.int32)
    batch, hist = idx.shape
    n = batch * hist
    assert batch % (NW * SGRP) == 0
    bpw = batch // NW
    assert bpw >= GBUF and bpw % SGRP == 0

    idx_p = jnp.pad(idx, ((0, 0), (0, PADW - hist))).reshape(-1)

    mesh = plsc.VectorSubcoreMesh(core_axis_name="c", subcore_axis_name="s")
    run = pl.kernel(
        functools.partial(_body, bpw=bpw, hist=hist),
        out_type=jax.ShapeDtypeStruct((n, PADW), jnp.float32),
        mesh=mesh,
        scratch_types=[
            pltpu.VMEM((bpw * PADW,), jnp.int32),
            pltpu.VMEM((D,), jnp.float32),
            pltpu.VMEM((GBUF, hist, D), jnp.float32),
            pltpu.VMEM((SBUF, SGRP * hist, PADW), jnp.float32),
            pltpu.SemaphoreType.DMA((GBUF,)),
            pltpu.SemaphoreType.DMA((SBUF,)),
        ],
        compiler_params=pltpu.CompilerParams(use_tc_tiling_on_sc=False),
    )
    res = run(weight, idx_p, bias)
    return res[:, :D]
